# Initial kernel scaffold; baseline (speedup 1.0000x reference)
import jax
import jax.numpy as jnp
from jax.experimental import pallas as pl


def kernel(weights):
    raise NotImplementedError("write your pallas kernel here")


# speedup vs baseline: 361.4588x; 361.4588x over previous
import jax
import jax.numpy as jnp
from jax import lax
from jax.experimental import pallas as pl
from jax.experimental.pallas import tpu as pltpu

_B = 8
_N = 64
_OFFS = ((-1, -1), (-1, 0), (-1, 1), (0, -1), (0, 1), (1, -1), (1, 0), (1, 1))


def _shift(a, dy, dx, fill):
    s = a
    if dy:
        s = jnp.roll(s, -dy, axis=1)
    if dx:
        s = jnp.roll(s, -dx, axis=2)
    yi = lax.broadcasted_iota(jnp.int32, (_B, _N, _N), 1)
    xi = lax.broadcasted_iota(jnp.int32, (_B, _N, _N), 2)
    valid = (yi + dy >= 0) & (yi + dy <= _N - 1) & (xi + dx >= 0) & (xi + dx <= _N - 1)
    return jnp.where(valid, s, fill)


def _dijkstra_kernel(w_ref, path_ref):
    w = w_ref[...]
    inf = jnp.float32(jnp.inf)
    flat = (lax.broadcasted_iota(jnp.int32, (_B, _N, _N), 1) * _N
            + lax.broadcasted_iota(jnp.int32, (_B, _N, _N), 2))
    dist0 = jnp.where(flat == 0, jnp.float32(0.0), inf)

    def sweep(dist):
        best = jnp.full((_B, _N, _N), inf, jnp.float32)
        for dy, dx in _OFFS:
            best = jnp.minimum(best, _shift(dist, dy, dx, inf))
        return jnp.minimum(dist, best + w)

    def bf_cond(c):
        _, changed, it = c
        return jnp.logical_and(changed, it < _N * _N)

    def bf_body(c):
        dist, _, it = c
        nd = sweep(dist)
        changed = jnp.any(nd < dist)
        return (nd, changed, it + 1)

    dist, _, _ = lax.while_loop(bf_cond, bf_body,
                                (dist0, jnp.bool_(True), jnp.int32(0)))

    best = jnp.full((_B, _N, _N), inf, jnp.float32)
    off = jnp.zeros((_B, _N, _N), jnp.int32)
    for dy, dx in _OFFS:
        nd = _shift(dist, dy, dx, inf)
        take = nd < best
        off = jnp.where(take, jnp.int32(dy * _N + dx), off)
        best = jnp.minimum(best, nd)
    pred = flat + off

    target = _N * _N - 1
    path0 = jnp.where(flat == target, jnp.float32(1.0), jnp.float32(0.0))
    cur0 = jnp.full((_B, 1, 1), target, jnp.int32)

    def bt_cond(c):
        cur, _, it = c
        return jnp.logical_and(jnp.any(cur != 0), it < _N * _N)

    def bt_body(c):
        cur, path, it = c
        active = cur != 0
        onehot = flat == cur
        nxt = jnp.sum(jnp.where(onehot, pred, 0), axis=(1, 2), keepdims=True)
        nxt = jnp.where(active, nxt, 0)
        write = (flat == nxt) & active
        path = jnp.where(write, jnp.float32(1.0), path)
        return (nxt, path, it + 1)

    _, path, _ = lax.while_loop(bt_cond, bt_body, (cur0, path0, jnp.int32(0)))
    path_ref[...] = path


def kernel(weights):
    return pl.pallas_call(
        _dijkstra_kernel,
        out_shape=jax.ShapeDtypeStruct((_B, _N, _N), jnp.float32),
    )(weights)
